# baseline (speedup 1.0000x reference)
import jax
import jax.numpy as jnp
from jax import lax
from jax.experimental import pallas as pl
from jax.experimental.pallas import tpu as pltpu
from jax.experimental.pallas import tpu_sc as plsc

HASH_N = 1000000
D = 64
B = 16384

_info = plsc.get_sparse_core_info()
NC, NS, L = _info.num_cores, _info.num_subcores, _info.num_lanes
NW = NC * NS
BPW = B // NW
CHUNK = 128
NCHUNK = BPW // CHUNK

PACK_ROWS = 4000
HALF_N = HASH_N // 2


def _pack_body(a_ref, b_ref, o_ref):
    o_ref[:, 0:D] = a_ref[...]
    o_ref[:, D:2 * D] = b_ref[...]


def _pack(table):
    nsteps = HALF_N // PACK_ROWS
    return pl.pallas_call(
        _pack_body,
        grid=(nsteps,),
        in_specs=[
            pl.BlockSpec((PACK_ROWS, D), lambda i: (i, 0)),
            pl.BlockSpec((PACK_ROWS, D), lambda i, n=nsteps: (i + n, 0)),
        ],
        out_specs=pl.BlockSpec((PACK_ROWS, 2 * D), lambda i: (i, 0)),
        out_shape=jax.ShapeDtypeStruct((HALF_N, 2 * D), jnp.float32),
    )(table, table)


def _body(tgt, ctx, tin2, tout2, out, idx_t, idx_c, pid_t, pid_c, rows_t, rows_c, res, sem):
    wid = lax.axis_index("s") * NC + lax.axis_index("c")
    base = wid * BPW

    pltpu.sync_copy(tgt.at[pl.ds(base, BPW)], idx_t)
    pltpu.sync_copy(ctx.at[pl.ds(base, BPW)], idx_c)

    def mod_body(j, _):
        sl = pl.ds(j * L, L)
        vt = lax.rem(idx_t[sl], HASH_N)
        vc = lax.rem(idx_c[sl], HASH_N)
        ht = jnp.where(vt >= HALF_N, jnp.int32(1), jnp.int32(0))
        hc = jnp.where(vc >= HALF_N, jnp.int32(1), jnp.int32(0))
        idx_t[sl] = ht * D
        idx_c[sl] = hc * D
        pid_t[sl] = vt - ht * HALF_N
        pid_c[sl] = vc - hc * HALF_N
        return 0

    lax.fori_loop(0, BPW // L, mod_body, 0)

    row_iota = lax.iota(jnp.int32, L)

    def chunk_body(n, _):
        cb = n * CHUNK
        sl = pl.ds(cb, CHUNK)
        ct = pltpu.async_copy(tin2.at[pid_t.at[sl]], rows_t, sem)
        cc = pltpu.async_copy(tout2.at[pid_c.at[sl]], rows_c, sem)
        ct.wait()
        cc.wait()

        def dot_group(g, _):
            rb = g * L
            tvec = idx_t[pl.ds(cb + rb, L)]
            cvec = idx_c[pl.ds(cb + rb, L)]
            tot = jnp.zeros((L,), jnp.float32)
            for i in range(L):
                r = rb + i
                ht = tvec[i]
                hc = cvec[i]
                acc = rows_t[r, pl.ds(ht, L)] * rows_c[r, pl.ds(hc, L)]
                for k in range(1, D // L):
                    acc = acc + (rows_t[r, pl.ds(ht + k * L, L)]
                                 * rows_c[r, pl.ds(hc + k * L, L)])
                tot = jnp.where(row_iota == i, jnp.sum(acc), tot)
            res[pl.ds(cb + rb, L)] = tot
            return 0

        lax.fori_loop(0, CHUNK // L, dot_group, 0)
        return 0

    lax.fori_loop(0, NCHUNK, chunk_body, 0)

    pltpu.sync_copy(res, out.at[pl.ds(base, BPW)])


def kernel(target, context, in_embed, out_embed):
    tin2 = jnp.concatenate([in_embed[:HALF_N], in_embed[HALF_N:]], axis=1)
    tout2 = jnp.concatenate([out_embed[:HALF_N], out_embed[HALF_N:]], axis=1)
    k = pl.kernel(
        _body,
        out_type=jax.ShapeDtypeStruct((B,), jnp.float32),
        mesh=plsc.VectorSubcoreMesh(core_axis_name="c", subcore_axis_name="s"),
        compiler_params=pltpu.CompilerParams(needs_layout_passes=False),
        scratch_types=[
            pltpu.VMEM((BPW,), jnp.int32),
            pltpu.VMEM((BPW,), jnp.int32),
            pltpu.VMEM((BPW,), jnp.int32),
            pltpu.VMEM((BPW,), jnp.int32),
            pltpu.VMEM((CHUNK, 2 * D), jnp.float32),
            pltpu.VMEM((CHUNK, 2 * D), jnp.float32),
            pltpu.VMEM((BPW,), jnp.float32),
            pltpu.SemaphoreType.DMA,
        ],
    )
    return k(target, context, tin2, tout2)


# speedup vs baseline: 2.1215x; 2.1215x over previous
import jax
import jax.numpy as jnp
from jax import lax
from jax.experimental import pallas as pl
from jax.experimental.pallas import tpu as pltpu
from jax.experimental.pallas import tpu_sc as plsc

HASH_N = 1000000
D = 64
B = 16384

_info = plsc.get_sparse_core_info()
NC, NS, L = _info.num_cores, _info.num_subcores, _info.num_lanes
NW = NC * NS
BPW = B // NW
CH = 128
NCH = BPW // CH
KSEM = 4
CPS = CH // KSEM
DRAIN_ROWS = 2 * CPS


def _body(tgt, ctx, tin, tout, out, idx_t, idx_c, rows_t, rows_c, res, sem):
    wid = lax.axis_index("s") * NC + lax.axis_index("c")
    base = wid * BPW

    pltpu.sync_copy(tgt.at[pl.ds(base, BPW)], idx_t)
    pltpu.sync_copy(ctx.at[pl.ds(base, BPW)], idx_c)

    def mod_body(j, _):
        sl = pl.ds(j * L, L)
        idx_t[sl] = lax.rem(idx_t[sl], HASH_N)
        idx_c[sl] = lax.rem(idx_c[sl], HASH_N)
        return 0

    lax.fori_loop(0, BPW // L, mod_body, 0)

    row_iota = lax.iota(jnp.int32, L)

    def fire(n):
        cb = n * CH
        g = n & 1
        for q in range(CH // L):
            tvec = idx_t[pl.ds(cb + q * L, L)]
            cvec = idx_c[pl.ds(cb + q * L, L)]
            for i in range(L):
                j = q * L + i
                k = j % KSEM
                pltpu.async_copy(
                    tin.at[pl.ds(tvec[i], 1)],
                    rows_t.at[g].at[pl.ds(j, 1)],
                    sem.at[g, k],
                )
                pltpu.async_copy(
                    tout.at[pl.ds(cvec[i], 1)],
                    rows_c.at[g].at[pl.ds(j, 1)],
                    sem.at[g, k],
                )

    def drain(m):
        g = m & 1
        for k in range(KSEM):
            pltpu.make_async_copy(
                tin.at[pl.ds(0, DRAIN_ROWS)],
                rows_t.at[0].at[pl.ds(0, DRAIN_ROWS)],
                sem.at[g, k],
            ).wait()

    def compute(m):
        cb = m * CH
        g = m & 1

        def dot_group(gg, _):
            rb = gg * L
            tot = jnp.zeros((L,), jnp.float32)
            for i in range(L):
                r = rb + i
                acc = rows_t[g, r, pl.ds(0, L)] * rows_c[g, r, pl.ds(0, L)]
                for k in range(1, D // L):
                    acc = acc + (rows_t[g, r, pl.ds(k * L, L)]
                                 * rows_c[g, r, pl.ds(k * L, L)])
                tot = jnp.where(row_iota == i, jnp.sum(acc), tot)
            res[pl.ds(cb + rb, L)] = tot
            return 0

        lax.fori_loop(0, CH // L, dot_group, 0)

    fire(0)

    def pipe_body(n, _):
        fire(n)
        drain(n - 1)
        compute(n - 1)
        return 0

    lax.fori_loop(1, NCH, pipe_body, 0)
    drain(NCH - 1)
    compute(NCH - 1)

    pltpu.sync_copy(res, out.at[pl.ds(base, BPW)])


def kernel(target, context, in_embed, out_embed):
    k = pl.kernel(
        _body,
        out_type=jax.ShapeDtypeStruct((B,), jnp.float32),
        mesh=plsc.VectorSubcoreMesh(core_axis_name="c", subcore_axis_name="s"),
        compiler_params=pltpu.CompilerParams(needs_layout_passes=False),
        scratch_types=[
            pltpu.VMEM((BPW,), jnp.int32),
            pltpu.VMEM((BPW,), jnp.int32),
            pltpu.VMEM((2, CH, D), jnp.float32),
            pltpu.VMEM((2, CH, D), jnp.float32),
            pltpu.VMEM((BPW,), jnp.float32),
            pltpu.SemaphoreType.DMA((2, KSEM)),
        ],
    )
    return k(target, context, in_embed, out_embed)
